# baseline (device time: 177089 ns/iter reference)
import jax
import jax.numpy as jnp
from jax import lax
from jax.experimental import pallas as pl
from jax.experimental.pallas import tpu as pltpu

N_DEV = 8
SQ = 2048
SKV = 2048
D_MODEL = 1024
DH = 128
H_LOCAL = 8
WIN = 128
QBLK = 256
KSPAN = 512
SCALE = 0.08838834764831843

_GROUPS = (
    (0, 768, (0, 1, 2)),
    (768, 640, (1, 2, 0)),
    (1408, 640, (2, 0, 1)),
)
_NG = len(_GROUPS)
_COMM_OFF = []
_off = 0
for _base, _rows, _order in _GROUPS:
    _offs = []
    for _s in range(3):
        _offs.append(_off)
        _off += _rows >> (_s + 1)
    _COMM_OFF.append(tuple(_offs))
_COMM_ROWS = _off


def _allreduce(out_ref, comm_ref, rs_send, rs_recv, ag_send, ag_recv):
    pos = lax.axis_index("i")
    q = lax.rem(pos, 4)
    zb = pos // 4
    xb = lax.rem((q + 1) // 2, 2)
    yb = q // 2
    partners = [
        zb * 4 + jnp.bitwise_xor(q, 1),
        zb * 4 + (3 - q),
        lax.rem(pos + 4, 8),
    ]
    bits = [xb, yb, zb]

    keep = [jnp.int32(g[0]) for g in _GROUPS]
    for s in range(3):
        rdmas = []
        for gi, (base, rows, order) in enumerate(_GROUPS):
            size = rows >> (s + 1)
            d = order[s]
            b = bits[d]
            send_start = keep[gi] + (1 - b) * size
            keep[gi] = keep[gi] + b * size
            rdma = pltpu.make_async_remote_copy(
                src_ref=out_ref.at[pl.ds(send_start, size), :],
                dst_ref=comm_ref.at[pl.ds(_COMM_OFF[gi][s], size), :],
                send_sem=rs_send.at[gi, s],
                recv_sem=rs_recv.at[gi, s],
                device_id=(partners[d],),
                device_id_type=pl.DeviceIdType.MESH,
            )
            rdma.start()
            rdmas.append(rdma)
        for gi, (base, rows, order) in enumerate(_GROUPS):
            size = rows >> (s + 1)
            rdmas[gi].wait_recv()
            out_ref[pl.ds(keep[gi], size), :] += comm_ref[
                pl.ds(_COMM_OFF[gi][s], size), :
            ]
            rdmas[gi].wait_send()

    cur = keep
    for s in (2, 1, 0):
        rdmas = []
        for gi, (base, rows, order) in enumerate(_GROUPS):
            size = rows >> (s + 1)
            rdma = pltpu.make_async_remote_copy(
                src_ref=out_ref.at[pl.ds(cur[gi], size), :],
                dst_ref=out_ref.at[pl.ds(cur[gi], size), :],
                send_sem=ag_send.at[gi, s],
                recv_sem=ag_recv.at[gi, s],
                device_id=(partners[order[s]],),
                device_id_type=pl.DeviceIdType.MESH,
            )
            rdma.start()
            rdmas.append(rdma)
        for gi, (base, rows, order) in enumerate(_GROUPS):
            size = rows >> (s + 1)
            rdmas[gi].wait_recv()
            rdmas[gi].wait_send()
            cur[gi] = cur[gi] - bits[order[s]] * size


def _body(
    x_ref, wq_ref, k_ref, v_ref, wo_ref, out_ref,
    xbf_scr, q_scr, ctx_scr, comm_ref,
    rs_send, rs_recv, ag_send, ag_recv,
):
    h = pl.program_id(0)

    @pl.when(h == 0)
    def _():
        xbf_scr[...] = x_ref[...].astype(jnp.bfloat16)

    q_scr[...] = (
        jnp.dot(
            xbf_scr[...],
            wq_ref[...].astype(jnp.bfloat16),
            preferred_element_type=jnp.float32,
        )
        * SCALE
    )

    def qblock(qb, carry):
        start = jnp.clip(qb * 2 - 1, 0, (SKV - KSPAN) // 128) * 128
        qblk = q_scr[pl.ds(qb * QBLK, QBLK), :].astype(jnp.bfloat16)
        kblk = k_ref[pl.ds(start, KSPAN), :].astype(jnp.bfloat16)
        s = lax.dot_general(
            qblk, kblk, (((1,), (1,)), ((), ())),
            preferred_element_type=jnp.float32,
        )
        qi = qb * QBLK + lax.broadcasted_iota(jnp.int32, (QBLK, KSPAN), 0)
        ki = start + lax.broadcasted_iota(jnp.int32, (QBLK, KSPAN), 1)
        mask = jnp.abs(qi - ki) <= WIN
        s = jnp.where(mask, s, -1e9)
        m = jnp.max(s, axis=1, keepdims=True)
        w = jnp.exp(s - m)
        denom = jnp.sum(w, axis=1, keepdims=True)
        vblk = v_ref[pl.ds(start, KSPAN), :].astype(jnp.bfloat16)
        ctx = jnp.dot(
            w.astype(jnp.bfloat16), vblk, preferred_element_type=jnp.float32
        )
        ctx_scr[pl.ds(qb * QBLK, QBLK), :] = ctx / denom
        return carry

    lax.fori_loop(0, SQ // QBLK, qblock, 0)

    contrib = jnp.dot(
        ctx_scr[...].astype(jnp.bfloat16),
        wo_ref[...].astype(jnp.bfloat16),
        preferred_element_type=jnp.float32,
    )

    @pl.when(h == 0)
    def _():
        out_ref[...] = contrib

    @pl.when(h != 0)
    def _():
        out_ref[...] += contrib

    @pl.when(h == H_LOCAL - 1)
    def _():
        _allreduce(out_ref, comm_ref, rs_send, rs_recv, ag_send, ag_recv)


def kernel(x, Wq, K_ext, V_ext, Wo):
    pos = lax.axis_index("i")
    K = lax.dynamic_slice_in_dim(
        K_ext[0], pos * H_LOCAL, H_LOCAL, axis=1
    ).reshape(SKV, H_LOCAL * DH)
    V = lax.dynamic_slice_in_dim(
        V_ext[0], pos * H_LOCAL, H_LOCAL, axis=1
    ).reshape(SKV, H_LOCAL * DH)

    out = pl.pallas_call(
        _body,
        grid=(H_LOCAL,),
        in_specs=[
            pl.BlockSpec((SQ, D_MODEL), lambda h: (0, 0)),
            pl.BlockSpec((D_MODEL, DH), lambda h: (0, h)),
            pl.BlockSpec((SKV, DH), lambda h: (0, h)),
            pl.BlockSpec((SKV, DH), lambda h: (0, h)),
            pl.BlockSpec((DH, D_MODEL), lambda h: (h, 0)),
        ],
        out_specs=pl.BlockSpec((SQ, D_MODEL), lambda h: (0, 0)),
        out_shape=jax.ShapeDtypeStruct((SQ, D_MODEL), jnp.float32),
        scratch_shapes=[
            pltpu.VMEM((SQ, D_MODEL), jnp.bfloat16),
            pltpu.VMEM((SQ, DH), jnp.float32),
            pltpu.VMEM((SQ, DH), jnp.float32),
            pltpu.VMEM((_COMM_ROWS, D_MODEL), jnp.float32),
            pltpu.SemaphoreType.DMA((_NG, 3)),
            pltpu.SemaphoreType.DMA((_NG, 3)),
            pltpu.SemaphoreType.DMA((_NG, 3)),
            pltpu.SemaphoreType.DMA((_NG, 3)),
        ],
        compiler_params=pltpu.CompilerParams(
            dimension_semantics=("arbitrary",),
            has_side_effects=True,
            vmem_limit_bytes=56 * 1024 * 1024,
        ),
    )(x[0], Wq, K, V, Wo)
    return out[None]
